# Initial kernel scaffold; baseline (speedup 1.0000x reference)
import jax
import jax.numpy as jnp
from jax.experimental import pallas as pl


def kernel(x, user_table, joke_table, W_out, b_out):
    raise NotImplementedError("write your pallas kernel here")


# speedup vs baseline: 259.5979x; 259.5979x over previous
import functools

import jax
import jax.numpy as jnp
from jax import lax
from jax.experimental import pallas as pl
from jax.experimental.pallas import tpu as pltpu
from jax.experimental.pallas import tpu_sc as plsc

_L = 16
_WORKERS = 32
_BLK_ROWS = 16
_OUT_W = 3 * _L


def _make_sc_kernel(B, K, NT):
  ROW_W = 2 * K
  RPW = B // _WORKERS
  NBLK = RPW // _BLK_ROWS
  BLK_W = _BLK_ROWS * ROW_W
  nsteps = K // _L
  rem = K - nsteps * _L

  mesh = plsc.VectorSubcoreMesh(core_axis_name="c", subcore_axis_name="s")

  @functools.partial(
      pl.kernel,
      out_type=jax.ShapeDtypeStruct((B * _OUT_W,), jnp.float32),
      mesh=mesh,
      compiler_params=pltpu.CompilerParams(needs_layout_passes=False),
      scratch_types=[
          pltpu.VMEM((BLK_W + _L,), jnp.int32),
          pltpu.VMEM((_BLK_ROWS * _OUT_W,), jnp.float32),
          pltpu.VMEM((NT,), jnp.float32),
          pltpu.VMEM((NT,), jnp.float32),
          pltpu.VMEM((NT,), jnp.float32),
          pltpu.VMEM((NT,), jnp.float32),
          pltpu.VMEM((NT,), jnp.float32),
          pltpu.VMEM((NT,), jnp.float32),
      ],
  )
  def sc(x_ref, u0_ref, u1_ref, u2_ref, j0_ref, j1_ref, j2_ref, out_ref,
         xbuf, obuf, tu0, tu1, tu2, tj0, tj1, tj2):
    wid = lax.axis_index("s") * 2 + lax.axis_index("c")
    base_word = wid * (RPW * ROW_W)

    pltpu.sync_copy(u0_ref, tu0)
    pltpu.sync_copy(u1_ref, tu1)
    pltpu.sync_copy(u2_ref, tu2)
    pltpu.sync_copy(j0_ref, tj0)
    pltpu.sync_copy(j1_ref, tj1)
    pltpu.sync_copy(j2_ref, tj2)

    xbuf[pl.ds(BLK_W, _L)] = jnp.zeros((_L,), jnp.int32)

    tail_mask = lax.iota(jnp.int32, _L) < rem
    zf = jnp.zeros((_L,), jnp.float32)

    def gathers(ui, ji):
      u0 = plsc.load_gather(tu0, [ui])
      u1 = plsc.load_gather(tu1, [ui])
      u2 = plsc.load_gather(tu2, [ui])
      j0 = plsc.load_gather(tj0, [ji])
      j1 = plsc.load_gather(tj1, [ji])
      j2 = plsc.load_gather(tj2, [ji])
      return u0, u1, u2, j0, j1, j2

    for g in range(NBLK):
      pltpu.sync_copy(
          x_ref.at[pl.ds(base_word + g * BLK_W, BLK_W)],
          xbuf.at[pl.ds(0, BLK_W)],
      )

      def row_body(r, carry):
        rbase = pl.multiple_of(r * ROW_W, 8)

        def step(s, acc):
          d, us, js = acc
          o = pl.multiple_of(rbase + s * _L, 8)
          ui = xbuf[pl.ds(o, _L)]
          ji = xbuf[pl.ds(o + K, _L)]
          u0, u1, u2, j0, j1, j2 = gathers(ui, ji)
          d = d + (u0 * j0 + u1 * j1 + u2 * j2)
          us = us + (u0 * u0 + u1 * u1 + u2 * u2)
          js = js + (j0 * j0 + j1 * j1 + j2 * j2)
          return (d, us, js)

        d, us, js = lax.fori_loop(0, nsteps, step, (zf, zf, zf))

        if rem:
          o = pl.multiple_of(rbase + nsteps * _L, 8)
          ui = xbuf[pl.ds(o, _L)]
          ji = xbuf[pl.ds(o + K, _L)]
          u0, u1, u2, j0, j1, j2 = gathers(ui, ji)
          d = d + jnp.where(tail_mask, u0 * j0 + u1 * j1 + u2 * j2, zf)
          us = us + jnp.where(tail_mask, u0 * u0 + u1 * u1 + u2 * u2, zf)
          js = js + jnp.where(tail_mask, j0 * j0 + j1 * j1 + j2 * j2, zf)

        ob = pl.multiple_of(r * _OUT_W, 8)
        obuf[pl.ds(ob, _L)] = d
        obuf[pl.ds(ob + _L, _L)] = us
        obuf[pl.ds(ob + 2 * _L, _L)] = js
        return carry

      lax.fori_loop(0, _BLK_ROWS, row_body, 0)

      oo = (wid * RPW + g * _BLK_ROWS) * _OUT_W
      pltpu.sync_copy(obuf, out_ref.at[pl.ds(oo, _BLK_ROWS * _OUT_W)])

  return sc


def _tc_head(p_ref, w_ref, b_ref, o_ref):
  p = p_ref[...]
  d = jnp.sum(p[:, 0:_L], axis=1, keepdims=True)
  us = jnp.sum(p[:, _L:2 * _L], axis=1, keepdims=True)
  js = jnp.sum(p[:, 2 * _L:3 * _L], axis=1, keepdims=True)
  inv = lax.rsqrt(jnp.maximum(us, 1e-12)) * lax.rsqrt(jnp.maximum(js, 1e-12))
  z = d * inv * w_ref[0, 0] + b_ref[0, 0]
  o_ref[...] = jax.nn.sigmoid(z)


def kernel(x, user_table, joke_table, W_out, b_out):
  B = x.shape[0]
  n_users = user_table.shape[0]
  K = x.shape[1] // 2
  NT = 1024

  ut = jnp.pad(user_table, ((0, NT - n_users), (0, 0)))
  jt = jnp.pad(joke_table, ((0, NT - joke_table.shape[0]), (0, 0)))

  sc = _make_sc_kernel(B, K, NT)
  partials = sc(
      x.reshape(-1),
      ut[:, 0], ut[:, 1], ut[:, 2],
      jt[:, 0], jt[:, 1], jt[:, 2],
  )
  p = partials.reshape(B, _OUT_W)

  out = pl.pallas_call(
      _tc_head,
      out_shape=jax.ShapeDtypeStruct((B, 1), jnp.float32),
  )(p, W_out, b_out.reshape(1, 1))
  return out


# speedup vs baseline: 318.2216x; 1.2258x over previous
import functools

import jax
import jax.numpy as jnp
from jax import lax
from jax.experimental import pallas as pl
from jax.experimental.pallas import tpu as pltpu
from jax.experimental.pallas import tpu_sc as plsc

_L = 16
_WORKERS = 32
_BLK_ROWS = 16
_OUT_W = 3 * _L


def _make_sc_kernel(B, K, NT):
  RPW = B // _WORKERS
  NBLK = RPW // _BLK_ROWS
  BLK_W = _BLK_ROWS * K
  nsteps = K // _L
  rem = K - nsteps * _L

  mesh = plsc.VectorSubcoreMesh(core_axis_name="c", subcore_axis_name="s")

  @functools.partial(
      pl.kernel,
      out_type=jax.ShapeDtypeStruct((B * _OUT_W,), jnp.float32),
      mesh=mesh,
      compiler_params=pltpu.CompilerParams(needs_layout_passes=False),
      scratch_types=[
          pltpu.VMEM((BLK_W,), jnp.int32),
          pltpu.VMEM((BLK_W,), jnp.int32),
          pltpu.VMEM((_BLK_ROWS * _OUT_W,), jnp.float32),
          pltpu.VMEM((NT,), jnp.float32),
          pltpu.VMEM((NT,), jnp.float32),
          pltpu.VMEM((NT,), jnp.float32),
          pltpu.VMEM((NT,), jnp.float32),
          pltpu.VMEM((NT,), jnp.float32),
          pltpu.VMEM((NT,), jnp.float32),
          pltpu.SemaphoreType.DMA,
          pltpu.SemaphoreType.DMA,
      ],
  )
  def sc(xp_ref, u0_ref, u1_ref, u2_ref, j0_ref, j1_ref, j2_ref, out_ref,
         xbuf0, xbuf1, obuf, tu0, tu1, tu2, tj0, tj1, tj2, sem0, sem1):
    wid = lax.axis_index("s") * 2 + lax.axis_index("c")
    base_word = wid * (RPW * K)

    pltpu.sync_copy(u0_ref, tu0)
    pltpu.sync_copy(u1_ref, tu1)
    pltpu.sync_copy(u2_ref, tu2)
    pltpu.sync_copy(j0_ref, tj0)
    pltpu.sync_copy(j1_ref, tj1)
    pltpu.sync_copy(j2_ref, tj2)

    xbufs = (xbuf0, xbuf1)
    sems = (sem0, sem1)

    def start_dma(g):
      return pltpu.async_copy(
          xp_ref.at[pl.ds(base_word + g * BLK_W, BLK_W)],
          xbufs[g % 2], sems[g % 2])

    dma0 = start_dma(0)

    tail_mask = lax.iota(jnp.int32, _L) >= (_L - rem)
    tail_col = K - _L
    zf = jnp.zeros((_L,), jnp.float32)
    lo_mask = jnp.full((_L,), 0xFFFF, jnp.int32)

    def pair_terms(w):
      ui = lax.bitwise_and(w, lo_mask)
      ji = lax.shift_right_logical(w, 16)
      u0 = plsc.load_gather(tu0, [ui])
      u1 = plsc.load_gather(tu1, [ui])
      u2 = plsc.load_gather(tu2, [ui])
      j0 = plsc.load_gather(tj0, [ji])
      j1 = plsc.load_gather(tj1, [ji])
      j2 = plsc.load_gather(tj2, [ji])
      t_d = u0 * j0 + u1 * j1 + u2 * j2
      t_u = u0 * u0 + u1 * u1 + u2 * u2
      t_j = j0 * j0 + j1 * j1 + j2 * j2
      return t_d, t_u, t_j

    pending = [dma0]
    for g in range(NBLK):
      pending.pop(0).wait()
      if g + 1 < NBLK:
        pending.append(start_dma(g + 1))
      xbuf = xbufs[g % 2]

      def row_body(r, carry):
        rbase = pl.multiple_of(r * K, 8)

        def step(s, acc):
          d, us, js = acc
          w = xbuf[pl.ds(pl.multiple_of(rbase + s * _L, 8), _L)]
          t_d, t_u, t_j = pair_terms(w)
          return (d + t_d, us + t_u, js + t_j)

        d, us, js = lax.fori_loop(0, nsteps, step, (zf, zf, zf))

        if rem:
          w = xbuf[pl.ds(rbase + tail_col, _L)]
          t_d, t_u, t_j = pair_terms(w)
          d = d + jnp.where(tail_mask, t_d, zf)
          us = us + jnp.where(tail_mask, t_u, zf)
          js = js + jnp.where(tail_mask, t_j, zf)

        ob = pl.multiple_of(r * _OUT_W, 8)
        obuf[pl.ds(ob, _L)] = d
        obuf[pl.ds(ob + _L, _L)] = us
        obuf[pl.ds(ob + 2 * _L, _L)] = js
        return carry

      lax.fori_loop(0, _BLK_ROWS, row_body, 0)

      oo = (wid * RPW + g * _BLK_ROWS) * _OUT_W
      pltpu.sync_copy(obuf, out_ref.at[pl.ds(oo, _BLK_ROWS * _OUT_W)])

  return sc


def _tc_head(p_ref, w_ref, b_ref, o_ref):
  p = p_ref[...]
  d = jnp.sum(p[:, 0:_L], axis=1, keepdims=True)
  us = jnp.sum(p[:, _L:2 * _L], axis=1, keepdims=True)
  js = jnp.sum(p[:, 2 * _L:3 * _L], axis=1, keepdims=True)
  inv = lax.rsqrt(jnp.maximum(us, 1e-12)) * lax.rsqrt(jnp.maximum(js, 1e-12))
  z = d * inv * w_ref[0, 0] + b_ref[0, 0]
  o_ref[...] = jax.nn.sigmoid(z)


def kernel(x, user_table, joke_table, W_out, b_out):
  B = x.shape[0]
  n_users = user_table.shape[0]
  K = x.shape[1] // 2
  NT = 1024

  xp = jnp.bitwise_or(x[:, :K], jnp.left_shift(x[:, K:], 16)).reshape(-1)

  ut = jnp.pad(user_table, ((0, NT - n_users), (0, 0)))
  jt = jnp.pad(joke_table, ((0, NT - joke_table.shape[0]), (0, 0)))

  sc = _make_sc_kernel(B, K, NT)
  p = sc(
      xp,
      ut[:, 0], ut[:, 1], ut[:, 2],
      jt[:, 0], jt[:, 1], jt[:, 2],
  ).reshape(B, _OUT_W)

  out = pl.pallas_call(
      _tc_head,
      out_shape=jax.ShapeDtypeStruct((B, 1), jnp.float32),
  )(p, W_out, b_out.reshape(1, 1))
  return out


# speedup vs baseline: 324.7104x; 1.0204x over previous
import functools

import jax
import jax.numpy as jnp
from jax import lax
from jax.experimental import pallas as pl
from jax.experimental.pallas import tpu as pltpu
from jax.experimental.pallas import tpu_sc as plsc

_L = 16
_WORKERS = 32
_BLK_ROWS = 16
_OUT_W = 3 * _L


def _make_sc_kernel(B, K, NT):
  RPW = B // _WORKERS
  NBLK = RPW // _BLK_ROWS
  BLK_W = _BLK_ROWS * K
  nsteps = K // _L
  rem = K - nsteps * _L

  mesh = plsc.VectorSubcoreMesh(core_axis_name="c", subcore_axis_name="s")

  @functools.partial(
      pl.kernel,
      out_type=jax.ShapeDtypeStruct((B * _OUT_W,), jnp.float32),
      mesh=mesh,
      compiler_params=pltpu.CompilerParams(needs_layout_passes=False),
      scratch_types=[
          pltpu.VMEM((BLK_W,), jnp.int32),
          pltpu.VMEM((BLK_W,), jnp.int32),
          pltpu.VMEM((_BLK_ROWS * _OUT_W,), jnp.float32),
          pltpu.VMEM((NT,), jnp.float32),
          pltpu.VMEM((NT,), jnp.float32),
          pltpu.VMEM((NT,), jnp.float32),
          pltpu.VMEM((NT,), jnp.float32),
          pltpu.VMEM((NT,), jnp.float32),
          pltpu.VMEM((NT,), jnp.float32),
          pltpu.SemaphoreType.DMA,
          pltpu.SemaphoreType.DMA,
      ],
  )
  def sc(xp_ref, u0_ref, u1_ref, u2_ref, j0_ref, j1_ref, j2_ref, out_ref,
         xbuf0, xbuf1, obuf, tu0, tu1, tu2, tj0, tj1, tj2, sem0, sem1):
    wid = lax.axis_index("s") * 2 + lax.axis_index("c")
    base_word = wid * (RPW * K)

    pltpu.sync_copy(u0_ref, tu0)
    pltpu.sync_copy(u1_ref, tu1)
    pltpu.sync_copy(u2_ref, tu2)
    pltpu.sync_copy(j0_ref, tj0)
    pltpu.sync_copy(j1_ref, tj1)
    pltpu.sync_copy(j2_ref, tj2)

    xbufs = (xbuf0, xbuf1)
    sems = (sem0, sem1)

    def start_dma(g):
      return pltpu.async_copy(
          xp_ref.at[pl.ds(base_word + g * BLK_W, BLK_W)],
          xbufs[g % 2], sems[g % 2])

    dma0 = start_dma(0)

    tail_mask = lax.iota(jnp.int32, _L) >= (_L - rem)
    tail_col = K - _L
    zf = jnp.zeros((_L,), jnp.float32)
    lo_mask = jnp.full((_L,), 0xFFFF, jnp.int32)

    def pair_terms(w):
      ui = lax.bitwise_and(w, lo_mask)
      ji = lax.shift_right_logical(w, 16)
      u0 = plsc.load_gather(tu0, [ui])
      u1 = plsc.load_gather(tu1, [ui])
      u2 = plsc.load_gather(tu2, [ui])
      j0 = plsc.load_gather(tj0, [ji])
      j1 = plsc.load_gather(tj1, [ji])
      j2 = plsc.load_gather(tj2, [ji])
      t_d = u0 * j0 + u1 * j1 + u2 * j2
      t_u = u0 * u0 + u1 * u1 + u2 * u2
      t_j = j0 * j0 + j1 * j1 + j2 * j2
      return t_d, t_u, t_j

    pending = [dma0]
    for g in range(NBLK):
      pending.pop(0).wait()
      if g + 1 < NBLK:
        pending.append(start_dma(g + 1))
      xbuf = xbufs[g % 2]

      def row_body(r, carry):
        rbase = pl.multiple_of(r * K, 8)

        d, us, js = zf, zf, zf
        for s in range(nsteps):
          w = xbuf[pl.ds(rbase + s * _L, _L)]
          t_d, t_u, t_j = pair_terms(w)
          d = d + t_d
          us = us + t_u
          js = js + t_j

        if rem:
          w = xbuf[pl.ds(rbase + tail_col, _L)]
          t_d, t_u, t_j = pair_terms(w)
          d = d + jnp.where(tail_mask, t_d, zf)
          us = us + jnp.where(tail_mask, t_u, zf)
          js = js + jnp.where(tail_mask, t_j, zf)

        ob = pl.multiple_of(r * _OUT_W, 8)
        obuf[pl.ds(ob, _L)] = d
        obuf[pl.ds(ob + _L, _L)] = us
        obuf[pl.ds(ob + 2 * _L, _L)] = js
        return carry

      lax.fori_loop(0, _BLK_ROWS, row_body, 0)

      oo = (wid * RPW + g * _BLK_ROWS) * _OUT_W
      pltpu.sync_copy(obuf, out_ref.at[pl.ds(oo, _BLK_ROWS * _OUT_W)])

  return sc


def _tc_head(p_ref, w_ref, b_ref, o_ref):
  p = p_ref[...]
  d = jnp.sum(p[:, 0:_L], axis=1, keepdims=True)
  us = jnp.sum(p[:, _L:2 * _L], axis=1, keepdims=True)
  js = jnp.sum(p[:, 2 * _L:3 * _L], axis=1, keepdims=True)
  inv = lax.rsqrt(jnp.maximum(us, 1e-12)) * lax.rsqrt(jnp.maximum(js, 1e-12))
  z = d * inv * w_ref[0, 0] + b_ref[0, 0]
  o_ref[...] = jax.nn.sigmoid(z)


def kernel(x, user_table, joke_table, W_out, b_out):
  B = x.shape[0]
  n_users = user_table.shape[0]
  K = x.shape[1] // 2
  NT = 1024

  xp = jnp.bitwise_or(x[:, :K], jnp.left_shift(x[:, K:], 16)).reshape(-1)

  ut = jnp.pad(user_table, ((0, NT - n_users), (0, 0)))
  jt = jnp.pad(joke_table, ((0, NT - joke_table.shape[0]), (0, 0)))

  sc = _make_sc_kernel(B, K, NT)
  p = sc(
      xp,
      ut[:, 0], ut[:, 1], ut[:, 2],
      jt[:, 0], jt[:, 1], jt[:, 2],
  ).reshape(B, _OUT_W)

  out = pl.pallas_call(
      _tc_head,
      out_shape=jax.ShapeDtypeStruct((B, 1), jnp.float32),
  )(p, W_out, b_out.reshape(1, 1))
  return out


# speedup vs baseline: 482.2785x; 1.4853x over previous
import functools

import jax
import jax.numpy as jnp
from jax import lax
from jax.experimental import pallas as pl
from jax.experimental.pallas import tpu as pltpu
from jax.experimental.pallas import tpu_sc as plsc

_L = 16
_WORKERS = 32
_LANE = 128
_SUB = 8
_GROUPS = _LANE // _L
_KB = 25


def _make_sc_kernel(B, K, NT):
  KTU = K // _SUB
  NCH = KTU // _KB
  OUT_PER_W = _GROUPS * 3 * _L

  mesh = plsc.VectorSubcoreMesh(core_axis_name="c", subcore_axis_name="s")

  @functools.partial(
      pl.kernel,
      out_type=jax.ShapeDtypeStruct((_WORKERS * OUT_PER_W,), jnp.float32),
      mesh=mesh,
      compiler_params=pltpu.CompilerParams(needs_layout_passes=False),
      scratch_types=[
          pltpu.VMEM((_KB, _SUB, _LANE), jnp.int32),
          pltpu.VMEM((_KB, _SUB, _LANE), jnp.int32),
          pltpu.VMEM((_KB, _SUB, _LANE), jnp.int32),
          pltpu.VMEM((_KB, _SUB, _LANE), jnp.int32),
          pltpu.VMEM((_GROUPS * 3 * _L,), jnp.float32),
          pltpu.VMEM((NT,), jnp.float32),
          pltpu.VMEM((NT,), jnp.float32),
          pltpu.VMEM((NT,), jnp.float32),
          pltpu.VMEM((NT,), jnp.float32),
          pltpu.VMEM((NT,), jnp.float32),
          pltpu.VMEM((NT,), jnp.float32),
          pltpu.SemaphoreType.DMA,
          pltpu.SemaphoreType.DMA,
          pltpu.SemaphoreType.DMA,
          pltpu.SemaphoreType.DMA,
      ],
  )
  def sc(xv_ref, u0_ref, u1_ref, u2_ref, j0_ref, j1_ref, j2_ref, out_ref,
         ubuf0, ubuf1, jbuf0, jbuf1, obuf, tu0, tu1, tu2, tj0, tj1, tj2,
         semu0, semu1, semj0, semj1):
    wid = lax.axis_index("s") * 2 + lax.axis_index("c")

    pltpu.sync_copy(u0_ref, tu0)
    pltpu.sync_copy(u1_ref, tu1)
    pltpu.sync_copy(u2_ref, tu2)
    pltpu.sync_copy(j0_ref, tj0)
    pltpu.sync_copy(j1_ref, tj1)
    pltpu.sync_copy(j2_ref, tj2)

    ubufs = (ubuf0, ubuf1)
    jbufs = (jbuf0, jbuf1)
    semus = (semu0, semu1)
    semjs = (semj0, semj1)

    def start(c, i):
      du = pltpu.async_copy(
          xv_ref.at[pl.ds(c * _KB, _KB), wid, :, :], ubufs[i], semus[i])
      dj = pltpu.async_copy(
          xv_ref.at[pl.ds(KTU + c * _KB, _KB), wid, :, :], jbufs[i], semjs[i])
      return du, dj

    pend = start(0, 0)
    zf = jnp.zeros((_L,), jnp.float32)

    for c in range(NCH):
      du, dj = pend
      du.wait()
      dj.wait()
      if c + 1 < NCH:
        pend = start(c + 1, (c + 1) % 2)
      ub = ubufs[c % 2]
      jb = jbufs[c % 2]

      for g in range(_GROUPS):
        gcol = g * _L

        def kti_body(t, acc):
          d, us, js = acc
          for ks in range(_SUB):
            ui = ub[t, ks, pl.ds(gcol, _L)]
            ji = jb[t, ks, pl.ds(gcol, _L)]
            u0 = plsc.load_gather(tu0, [ui])
            u1 = plsc.load_gather(tu1, [ui])
            u2 = plsc.load_gather(tu2, [ui])
            j0 = plsc.load_gather(tj0, [ji])
            j1 = plsc.load_gather(tj1, [ji])
            j2 = plsc.load_gather(tj2, [ji])
            d = d + (u0 * j0 + u1 * j1 + u2 * j2)
            us = us + (u0 * u0 + u1 * u1 + u2 * u2)
            js = js + (j0 * j0 + j1 * j1 + j2 * j2)
          return (d, us, js)

        d, us, js = lax.fori_loop(0, _KB, kti_body, (zf, zf, zf))

        ob = g * 3 * _L
        if c == 0:
          obuf[pl.ds(ob, _L)] = d
          obuf[pl.ds(ob + _L, _L)] = us
          obuf[pl.ds(ob + 2 * _L, _L)] = js
        else:
          obuf[pl.ds(ob, _L)] = obuf[pl.ds(ob, _L)] + d
          obuf[pl.ds(ob + _L, _L)] = obuf[pl.ds(ob + _L, _L)] + us
          obuf[pl.ds(ob + 2 * _L, _L)] = obuf[pl.ds(ob + 2 * _L, _L)] + js

    pltpu.sync_copy(obuf, out_ref.at[pl.ds(wid * OUT_PER_W, OUT_PER_W)])

  return sc


def _tc_head(p_ref, w_ref, b_ref, o_ref):
  p = p_ref[...]
  d = p[:, 0:_L]
  us = p[:, _L:2 * _L]
  js = p[:, 2 * _L:3 * _L]
  inv = lax.rsqrt(jnp.maximum(us, 1e-12)) * lax.rsqrt(jnp.maximum(js, 1e-12))
  z = d * inv * w_ref[0, 0] + b_ref[0, 0]
  o_ref[...] = jax.nn.sigmoid(z)


def kernel(x, user_table, joke_table, W_out, b_out):
  B = x.shape[0]
  n_users = user_table.shape[0]
  K2 = x.shape[1]
  K = K2 // 2
  NT = 1024

  xv = x.reshape(B // _LANE, _LANE, K2 // _SUB, _SUB).transpose(2, 0, 3, 1)

  ut = jnp.pad(user_table, ((0, NT - n_users), (0, 0)))
  jt = jnp.pad(joke_table, ((0, NT - joke_table.shape[0]), (0, 0)))

  sc = _make_sc_kernel(B, K, NT)
  p = sc(
      xv,
      ut[:, 0], ut[:, 1], ut[:, 2],
      jt[:, 0], jt[:, 1], jt[:, 2],
  ).reshape(B // _L, 3 * _L)

  out = pl.pallas_call(
      _tc_head,
      out_shape=jax.ShapeDtypeStruct((B // _L, _L), jnp.float32),
  )(p, W_out, b_out.reshape(1, 1))
  return out.reshape(B, 1)
